# baseline (speedup 1.0000x reference)
import jax
import jax.numpy as jnp
from jax.experimental import pallas as pl

N = 10000
NCLASS = 16
HID = 128


def _gcn_conv(x, src, dst, ew, W, b, num_nodes):
    loop = jnp.arange(num_nodes, dtype=src.dtype)
    src2 = jnp.concatenate([src, loop])
    dst2 = jnp.concatenate([dst, loop])
    ew2 = jnp.concatenate([ew, jnp.ones((num_nodes,), dtype=ew.dtype)])
    deg = jnp.zeros((num_nodes,), dtype=ew.dtype).at[dst2].add(ew2)
    dinv = jnp.where(deg > 0, deg ** -0.5, 0.0)
    norm = dinv[src2] * ew2 * dinv[dst2]
    h = x @ W
    out = jnp.zeros((num_nodes, W.shape[1]), dtype=h.dtype).at[dst2].add(norm[:, None] * h[src2])
    return out + b


def _fc_kernel(h_ref, w_ref, b_ref, o_ref):
    o_ref[...] = h_ref[...] @ w_ref[...] + b_ref[...]


def kernel(x, edge_index, edge_weight, W1, b1, W2, b2, Wfc, bfc):
    src = edge_index[0]
    dst = edge_index[1]
    h = _gcn_conv(x, src, dst, edge_weight, W1, b1, N)
    h = jax.nn.relu(h)
    h = _gcn_conv(h, src, dst, edge_weight, W2, b2, N)
    h = jax.nn.relu(h)
    out = pl.pallas_call(
        _fc_kernel,
        out_shape=jax.ShapeDtypeStruct((N, NCLASS), jnp.float32),
    )(h, Wfc, bfc[None, :])
    return out


# speedup vs baseline: 10.5465x; 10.5465x over previous
import functools

import jax
import jax.numpy as jnp
from jax import lax
from jax.experimental import pallas as pl
from jax.experimental.pallas import tpu as pltpu
from jax.experimental.pallas import tpu_sc as plsc

N = 10000
NP = 10240
E = 640000
NCLASS = 16
HID = 128

NC = 2
NS = 16
NW = NC * NS

B = 80
E2 = 655360
EPT = E2 // NW
RPT = EPT // B
ROWS = E2 // B
NPT = NP // NS
DCH = 2048
SR = 16

_mesh = plsc.VectorSubcoreMesh(core_axis_name="c", subcore_axis_name="s")
_sc_params = pltpu.CompilerParams(needs_layout_passes=False)


@functools.partial(
    pl.kernel,
    out_type=jax.ShapeDtypeStruct((NW * NP,), jnp.float32),
    mesh=_mesh,
    compiler_params=_sc_params,
    scratch_types=[
        pltpu.VMEM((DCH,), jnp.int32),
        pltpu.VMEM((DCH,), jnp.float32),
        pltpu.VMEM((NP,), jnp.float32),
        pltpu.SemaphoreType.DMA,
    ],
)
def _deg_kernel(dst_hbm, ew_hbm, out_hbm, dst_v, ew_v, deg_v, sem):
    cid = lax.axis_index("c")
    sid = lax.axis_index("s")
    w = cid * NS + sid
    base = w * EPT
    zero = jnp.zeros((16,), jnp.float32)

    def zbody(i, _):
        deg_v[pl.ds(i * 16, 16)] = zero
        return 0

    lax.fori_loop(0, NP // 16, zbody, 0)

    def body(i, _):
        d = dst_v[pl.ds(i * 16, 16)]
        e = ew_v[pl.ds(i * 16, 16)]
        plsc.addupdate_scatter(deg_v, [d], e)
        return 0

    for c in range(EPT // DCH):
        cp1 = pltpu.async_copy(dst_hbm.at[pl.ds(base + c * DCH, DCH)], dst_v, sem)
        cp2 = pltpu.async_copy(ew_hbm.at[pl.ds(base + c * DCH, DCH)], ew_v, sem)
        cp1.wait()
        cp2.wait()
        lax.fori_loop(0, DCH // 16, body, 0)
    pltpu.sync_copy(deg_v, out_hbm.at[pl.ds(w * NP, NP)])


@functools.partial(
    pl.kernel,
    out_type=jax.ShapeDtypeStruct((ROWS, B), jnp.float32),
    mesh=_mesh,
    compiler_params=_sc_params,
    scratch_types=[
        pltpu.VMEM((SR, B), jnp.int32),
        pltpu.VMEM((SR, B), jnp.int32),
        pltpu.VMEM((SR, B), jnp.float32),
        pltpu.VMEM((SR, B), jnp.float32),
        pltpu.VMEM((NP,), jnp.float32),
        pltpu.SemaphoreType.DMA,
    ],
)
def _norm_kernel(src_hbm, dst_hbm, ew_hbm, dinv_hbm, nrm_out,
                 src_v, dst_v, ew_v, nrm_v, dinv_v, sem):
    cid = lax.axis_index("c")
    sid = lax.axis_index("s")
    w = cid * NS + sid
    r0 = w * RPT
    pltpu.sync_copy(dinv_hbm, dinv_v)

    def body(i, _):
        r = i // (B // 16)
        g = i % (B // 16)
        s = src_v[r, pl.ds(g * 16, 16)]
        d = dst_v[r, pl.ds(g * 16, 16)]
        e = ew_v[r, pl.ds(g * 16, 16)]
        nv = plsc.load_gather(dinv_v, [s]) * e * plsc.load_gather(dinv_v, [d])
        nrm_v[r, pl.ds(g * 16, 16)] = nv
        return 0

    for c in range(RPT // SR):
        rc = r0 + c * SR
        cps = [
            pltpu.async_copy(src_hbm.at[pl.ds(rc, SR)], src_v, sem),
            pltpu.async_copy(dst_hbm.at[pl.ds(rc, SR)], dst_v, sem),
            pltpu.async_copy(ew_hbm.at[pl.ds(rc, SR)], ew_v, sem),
        ]
        for cp in cps:
            cp.wait()
        lax.fori_loop(0, SR * (B // 16), body, 0)
        pltpu.sync_copy(nrm_v, nrm_out.at[pl.ds(rc, SR)])


@functools.partial(
    pl.kernel,
    out_type=jax.ShapeDtypeStruct((NC, NP, HID), jnp.float32),
    mesh=_mesh,
    compiler_params=_sc_params,
    scratch_types=[
        pltpu.VMEM((SR, B), jnp.int32),
        pltpu.VMEM((SR, B), jnp.int32),
        pltpu.VMEM((SR, B), jnp.float32),
        pltpu.VMEM((B, HID), jnp.float32),
        pltpu.VMEM_SHARED((NP, HID), jnp.float32),
        pltpu.SemaphoreType.DMA,
    ],
)
def _agg_kernel(src_hbm, dst_hbm, nrm_hbm, h_hbm, acc_out,
                src_v, dst_v, nrm_v, xb_v, acc_sh, sem):
    cid = lax.axis_index("c")
    sid = lax.axis_index("s")
    w = cid * NS + sid
    r0 = w * RPT
    zero = jnp.zeros((16,), jnp.float32)

    def zbody(i, _):
        def zcol(k, _):
            xb_v[i, pl.ds(k * 16, 16)] = zero
            return 0

        lax.fori_loop(0, HID // 16, zcol, 0)
        return 0

    lax.fori_loop(0, B, zbody, 0)
    for q in range(NPT // B):
        pltpu.sync_copy(xb_v, acc_sh.at[pl.ds(sid * NPT + q * B, B)])
    plsc.subcore_barrier()

    def row_body(i, _):
        pltpu.async_copy(h_hbm.at[src_v.at[i]], xb_v, sem).wait()

        def srow(j, _):
            bc = plsc.load_gather(
                nrm_v,
                [jnp.full((16,), i, jnp.int32), jnp.full((16,), j, jnp.int32)],
            )

            def scol(k, _):
                xb_v[j, pl.ds(k * 16, 16)] = xb_v[j, pl.ds(k * 16, 16)] * bc
                return 0

            lax.fori_loop(0, HID // 16, scol, 0)
            return 0

        lax.fori_loop(0, B, srow, 0)
        pltpu.async_copy(xb_v, acc_sh.at[dst_v.at[i]], sem, add=True).wait()
        return 0

    def chunk_body(c, _):
        rc = r0 + c * SR
        cps = [
            pltpu.async_copy(src_hbm.at[pl.ds(rc, SR)], src_v, sem),
            pltpu.async_copy(dst_hbm.at[pl.ds(rc, SR)], dst_v, sem),
            pltpu.async_copy(nrm_hbm.at[pl.ds(rc, SR)], nrm_v, sem),
        ]
        for cp in cps:
            cp.wait()
        lax.fori_loop(0, SR, row_body, 0)
        return 0

    lax.fori_loop(0, RPT // SR, chunk_body, 0)
    plsc.subcore_barrier()
    pltpu.sync_copy(acc_sh.at[pl.ds(sid * NPT, NPT)],
                    acc_out.at[cid, pl.ds(sid * NPT, NPT)])


def _dinv_body(degp_ref, dinv_ref):
    dinv_ref[...] = lax.rsqrt(jnp.maximum(jnp.sum(degp_ref[...], axis=0), 1e-12))


def _mm_body(x_ref, w_ref, o_ref):
    o_ref[...] = jnp.dot(x_ref[...], w_ref[...], preferred_element_type=jnp.float32)


def _bias_relu_body(a0_ref, a1_ref, b_ref, o_ref):
    o_ref[...] = jnp.maximum(a0_ref[...] + a1_ref[...] + b_ref[...], 0.0)


def _mm2_body(a0_ref, a1_ref, w_ref, b_ref, wfc_ref, bfc_ref, o_ref):
    agg = a0_ref[...] + a1_ref[...]
    h = jnp.maximum(
        jnp.dot(agg, w_ref[...], preferred_element_type=jnp.float32) + b_ref[...],
        0.0,
    )
    o_ref[...] = jnp.dot(h, wfc_ref[...], preferred_element_type=jnp.float32) + bfc_ref[...]


def kernel(x, edge_index, edge_weight, W1, b1, W2, b2, Wfc, bfc):
    src = edge_index[0].astype(jnp.int32)
    dst = edge_index[1].astype(jnp.int32)
    ew = edge_weight.astype(jnp.float32)

    pad = E2 - E - N
    loop = jnp.arange(N, dtype=jnp.int32)
    zpad = jnp.zeros((pad,), jnp.int32)
    src_e = jnp.concatenate([src, loop, zpad]).reshape(ROWS, B)
    dst_e = jnp.concatenate([dst, loop, zpad]).reshape(ROWS, B)
    ew_e = jnp.concatenate(
        [ew, jnp.ones((N,), jnp.float32), jnp.zeros((pad,), jnp.float32)]
    ).reshape(ROWS, B)

    deg_p = _deg_kernel(dst_e.reshape(E2), ew_e.reshape(E2))
    dinv = pl.pallas_call(
        _dinv_body,
        out_shape=jax.ShapeDtypeStruct((NP,), jnp.float32),
    )(deg_p.reshape(NW, NP))

    nrm = _norm_kernel(src_e, dst_e, ew_e, dinv)
    g0 = pl.pallas_call(
        _mm_body,
        out_shape=jax.ShapeDtypeStruct((N, HID), jnp.float32),
    )(x, W1)

    acc1 = _agg_kernel(src_e, dst_e, nrm, g0)
    h1 = pl.pallas_call(
        _bias_relu_body,
        out_shape=jax.ShapeDtypeStruct((NP, HID), jnp.float32),
    )(acc1[0], acc1[1], b1[None, :])

    acc2 = _agg_kernel(src_e, dst_e, nrm, h1)
    out = pl.pallas_call(
        _mm2_body,
        out_shape=jax.ShapeDtypeStruct((NP, NCLASS), jnp.float32),
    )(acc2[0], acc2[1], W2, b2[None, :], Wfc, bfc[None, :])
    return out[:N]


# speedup vs baseline: 14.5360x; 1.3783x over previous
import functools

import jax
import jax.numpy as jnp
from jax import lax
from jax.experimental import pallas as pl
from jax.experimental.pallas import tpu as pltpu
from jax.experimental.pallas import tpu_sc as plsc

N = 10000
NP = 10240
E = 640000
NCLASS = 16
HID = 128

NC = 2
NS = 16
NW = NC * NS

B = 80
E2 = 655360
EPT = E2 // NW
RPT = EPT // B
ROWS = E2 // B
NPT = NP // NS
DCH = 1280
SR = 16
NCHUNK = RPT // SR

_mesh = plsc.VectorSubcoreMesh(core_axis_name="c", subcore_axis_name="s")
_sc_params = pltpu.CompilerParams(needs_layout_passes=False)


@functools.partial(
    pl.kernel,
    out_type=jax.ShapeDtypeStruct((NW * NP,), jnp.float32),
    mesh=_mesh,
    compiler_params=_sc_params,
    scratch_types=[
        pltpu.VMEM((DCH,), jnp.int32),
        pltpu.VMEM((DCH,), jnp.float32),
        pltpu.VMEM((NP,), jnp.float32),
        pltpu.SemaphoreType.DMA,
    ],
)
def _deg_kernel(dst_hbm, ew_hbm, out_hbm, dst_v, ew_v, deg_v, sem):
    cid = lax.axis_index("c")
    sid = lax.axis_index("s")
    w = cid * NS + sid
    base = w * EPT
    zero = jnp.zeros((16,), jnp.float32)

    def zbody(i, _):
        deg_v[pl.ds(i * 16, 16)] = zero
        return 0

    lax.fori_loop(0, NP // 16, zbody, 0)

    def body(i, _):
        d = dst_v[pl.ds(i * 16, 16)]
        e = ew_v[pl.ds(i * 16, 16)]
        plsc.addupdate_scatter(deg_v, [d], e)
        return 0

    for c in range(EPT // DCH):
        cp1 = pltpu.async_copy(dst_hbm.at[pl.ds(base + c * DCH, DCH)], dst_v, sem)
        cp2 = pltpu.async_copy(ew_hbm.at[pl.ds(base + c * DCH, DCH)], ew_v, sem)
        cp1.wait()
        cp2.wait()
        lax.fori_loop(0, DCH // 16, body, 0)
    pltpu.sync_copy(deg_v, out_hbm.at[pl.ds(w * NP, NP)])


@functools.partial(
    pl.kernel,
    out_type=jax.ShapeDtypeStruct((NC, NP, HID), jnp.float32),
    mesh=_mesh,
    compiler_params=_sc_params,
    scratch_types=[
        pltpu.VMEM((SR, B), jnp.int32),
        pltpu.VMEM((SR, B), jnp.int32),
        pltpu.VMEM((SR, B), jnp.float32),
        pltpu.VMEM((SR, B), jnp.float32),
        pltpu.VMEM((NP,), jnp.float32),
        pltpu.VMEM((B, HID), jnp.float32),
        pltpu.VMEM((B, HID), jnp.float32),
        pltpu.VMEM_SHARED((NP, HID), jnp.float32),
        pltpu.SemaphoreType.DMA,
        pltpu.SemaphoreType.DMA,
    ],
)
def _agg_kernel(src_hbm, dst_hbm, ew_hbm, dinv_hbm, h_hbm, acc_out,
                src_v, dst_v, ew_v, nrm_v, dinv_v, xb0, xb1, acc_sh,
                sem_st, sem_g):
    cid = lax.axis_index("c")
    sid = lax.axis_index("s")
    w = cid * NS + sid
    r0 = w * RPT
    zero = jnp.zeros((16,), jnp.float32)
    bufs = (xb0, xb1)

    def zbody(i, _):
        for k in range(HID // 16):
            xb0[i, pl.ds(k * 16, 16)] = zero
        return 0

    lax.fori_loop(0, B, zbody, 0)
    for q in range(NPT // B):
        pltpu.sync_copy(xb0, acc_sh.at[pl.ds(sid * NPT + q * B, B)])
    plsc.subcore_barrier()

    pltpu.sync_copy(dinv_hbm, dinv_v)

    def gissue(r, buf):
        return pltpu.async_copy(h_hbm.at[src_v.at[r]], buf, sem_g)

    def gwait(r, buf):
        pltpu.make_async_copy(h_hbm.at[src_v.at[r]], buf, sem_g).wait()

    def scale_row(r, buf):
        def ebody(j, _):
            bc = plsc.load_gather(
                nrm_v,
                [jnp.full((16,), r, jnp.int32), jnp.full((16,), j, jnp.int32)],
            )
            for k in range(HID // 16):
                buf[j, pl.ds(k * 16, 16)] = buf[j, pl.ds(k * 16, 16)] * bc
            return 0

        lax.fori_loop(0, B, ebody, 0)

    def scatter_row(r, buf):
        pltpu.sync_copy(buf, acc_sh.at[dst_v.at[r]], add=True)

    def chunk_body(c, _):
        rc = r0 + c * SR
        cps = [
            pltpu.async_copy(src_hbm.at[pl.ds(rc, SR)], src_v, sem_st),
            pltpu.async_copy(dst_hbm.at[pl.ds(rc, SR)], dst_v, sem_st),
            pltpu.async_copy(ew_hbm.at[pl.ds(rc, SR)], ew_v, sem_st),
        ]
        for cp in cps:
            cp.wait()

        def nbody(g, _):
            r = g // (B // 16)
            gg = g % (B // 16)
            s = src_v[r, pl.ds(gg * 16, 16)]
            d = dst_v[r, pl.ds(gg * 16, 16)]
            e = ew_v[r, pl.ds(gg * 16, 16)]
            nv = plsc.load_gather(dinv_v, [s]) * e * plsc.load_gather(dinv_v, [d])
            nrm_v[r, pl.ds(gg * 16, 16)] = nv
            return 0

        lax.fori_loop(0, SR * (B // 16), nbody, 0)

        gissue(0, bufs[0])

        def pair_body(p, _):
            r = 2 * p
            gissue(r + 1, bufs[1])
            gwait(r, bufs[0])
            scale_row(r, bufs[0])
            scatter_row(r, bufs[0])

            @pl.when(r + 2 < SR)
            def _():
                gissue(r + 2, bufs[0])

            gwait(r + 1, bufs[1])
            scale_row(r + 1, bufs[1])
            scatter_row(r + 1, bufs[1])
            return 0

        lax.fori_loop(0, SR // 2, pair_body, 0)
        return 0

    lax.fori_loop(0, NCHUNK, chunk_body, 0)
    plsc.subcore_barrier()
    pltpu.sync_copy(acc_sh.at[pl.ds(sid * NPT, NPT)],
                    acc_out.at[cid, pl.ds(sid * NPT, NPT)])


def _dinv_body(degp_ref, dinv_ref):
    dinv_ref[...] = lax.rsqrt(jnp.maximum(jnp.sum(degp_ref[...], axis=0), 1e-12))


def _mm_body(x_ref, w_ref, o_ref):
    o_ref[...] = jnp.dot(x_ref[...], w_ref[...], preferred_element_type=jnp.float32)


def _bias_relu_body(a0_ref, a1_ref, b_ref, o_ref):
    o_ref[...] = jnp.maximum(a0_ref[...] + a1_ref[...] + b_ref[...], 0.0)


def _mm2_body(a0_ref, a1_ref, w_ref, b_ref, wfc_ref, bfc_ref, o_ref):
    agg = a0_ref[...] + a1_ref[...]
    h = jnp.maximum(
        jnp.dot(agg, w_ref[...], preferred_element_type=jnp.float32) + b_ref[...],
        0.0,
    )
    o_ref[...] = jnp.dot(h, wfc_ref[...], preferred_element_type=jnp.float32) + bfc_ref[...]


def kernel(x, edge_index, edge_weight, W1, b1, W2, b2, Wfc, bfc):
    src = edge_index[0].astype(jnp.int32)
    dst = edge_index[1].astype(jnp.int32)
    ew = edge_weight.astype(jnp.float32)

    pad = E2 - E - N
    loop = jnp.arange(N, dtype=jnp.int32)
    zpad = jnp.zeros((pad,), jnp.int32)
    src_e = jnp.concatenate([src, loop, zpad]).reshape(ROWS, B)
    dst_e = jnp.concatenate([dst, loop, zpad]).reshape(ROWS, B)
    ew_e = jnp.concatenate(
        [ew, jnp.ones((N,), jnp.float32), jnp.zeros((pad,), jnp.float32)]
    ).reshape(ROWS, B)

    deg_p = _deg_kernel(dst_e.reshape(E2), ew_e.reshape(E2))
    dinv = pl.pallas_call(
        _dinv_body,
        out_shape=jax.ShapeDtypeStruct((NP,), jnp.float32),
    )(deg_p.reshape(NW, NP))

    g0 = pl.pallas_call(
        _mm_body,
        out_shape=jax.ShapeDtypeStruct((N, HID), jnp.float32),
    )(x, W1)

    acc1 = _agg_kernel(src_e, dst_e, ew_e, dinv, g0)
    h1 = pl.pallas_call(
        _bias_relu_body,
        out_shape=jax.ShapeDtypeStruct((NP, HID), jnp.float32),
    )(acc1[0], acc1[1], b1[None, :])

    acc2 = _agg_kernel(src_e, dst_e, ew_e, dinv, h1)
    out = pl.pallas_call(
        _mm2_body,
        out_shape=jax.ShapeDtypeStruct((NP, NCLASS), jnp.float32),
    )(acc2[0], acc2[1], W2, b2[None, :], Wfc, bfc[None, :])
    return out[:N]
